# Initial kernel scaffold; baseline (speedup 1.0000x reference)
import functools

import jax
import jax.numpy as jnp
from jax import lax
from jax.experimental import pallas as pl
from jax.experimental.pallas import tpu as pltpu
from jax.experimental.pallas import tpu_sc as plsc

NC = 2
NS = 16
NW = NC * NS
LANES = 16
CHUNK = 128


@functools.lru_cache(maxsize=None)
def _make_seg_kernel(n_nodes, n_pad, h, nch, with_counts):
    ept = nch * CHUNK
    rpt = n_pad // NS
    mesh = plsc.VectorSubcoreMesh(
        core_axis_name="c", subcore_axis_name="s",
        num_cores=NC, num_subcores=NS)

    out_type = [jax.ShapeDtypeStruct((NC, n_pad, h), jnp.float32)]
    if with_counts:
        out_type.append(jax.ShapeDtypeStruct((NW, n_pad), jnp.float32))

    scratch = [
        pltpu.VMEM((nch, CHUNK), jnp.int32),
        pltpu.VMEM((nch, CHUNK), jnp.int32),
        pltpu.VMEM((CHUNK, h), jnp.float32),
        pltpu.VMEM_SHARED((n_pad, h), jnp.float32),
        pltpu.SemaphoreType.DMA,
    ]
    if with_counts:
        scratch.insert(2, pltpu.VMEM((ept,), jnp.int32))
        scratch.insert(3, pltpu.VMEM((n_pad,), jnp.float32))

    @functools.partial(pl.kernel, out_type=tuple(out_type), mesh=mesh,
                       scratch_types=tuple(scratch))
    def k(*refs):
        if with_counts:
            (y_hbm, src_hbm, dst_hbm, dstf_hbm, zeros_hbm,
             part_out, cnt_out,
             src_v, dst_v, dstf_v, cnt_v, rows_v, acc_sh, sem) = refs
        else:
            (y_hbm, src_hbm, dst_hbm, dstf_hbm, zeros_hbm,
             part_out,
             src_v, dst_v, rows_v, acc_sh, sem) = refs
        cid = lax.axis_index("c")
        sid = lax.axis_index("s")
        wid = sid * NC + cid

        pltpu.sync_copy(src_hbm.at[wid], src_v)
        pltpu.sync_copy(dst_hbm.at[wid], dst_v)
        pltpu.sync_copy(zeros_hbm, acc_sh.at[pl.ds(sid * rpt, rpt)])

        if with_counts:
            pltpu.sync_copy(dstf_hbm.at[wid], dstf_v)
            zero16 = jnp.zeros((LANES,), jnp.float32)

            def zbody(i, c):
                cnt_v[pl.ds(i * LANES, LANES)] = zero16
                return c
            lax.fori_loop(0, n_pad // LANES, zbody, 0)

            one16 = jnp.ones((LANES,), jnp.float32)

            def cbody(t, c):
                idx = dstf_v[pl.ds(t * LANES, LANES)]
                plsc.addupdate_scatter(cnt_v, [idx], one16)
                return c
            lax.fori_loop(0, ept // LANES, cbody, 0)

        plsc.subcore_barrier()

        def gbody(j, c):
            pltpu.async_copy(y_hbm.at[src_v.at[j]], rows_v, sem).wait()
            pltpu.sync_copy(rows_v, acc_sh.at[dst_v.at[j]], add=True)
            return c
        lax.fori_loop(0, nch, gbody, 0)

        plsc.subcore_barrier()
        pltpu.sync_copy(acc_sh.at[pl.ds(sid * rpt, rpt)],
                        part_out.at[cid, pl.ds(sid * rpt, rpt)])
        if with_counts:
            pltpu.sync_copy(cnt_v, cnt_out.at[wid])

    return k


def _tc1_body(x_ref, tau_ref, wl_ref, wr_ref, b_ref, yl_ref, yr_ref):
    xb = x_ref[...]
    t = tau_ref[...]
    d = xb.shape[1]
    yl_ref[...] = (jnp.dot(xb, wl_ref[:d, :], preferred_element_type=jnp.float32)
                   + t * wl_ref[d:d + 1, :])
    yr_ref[...] = (jnp.dot(xb, wr_ref[:d, :], preferred_element_type=jnp.float32)
                   + t * wr_ref[d:d + 1, :] + b_ref[...])


def _tc2_body(p_ref, cnt_ref, yr_ref, wl_ref, wr_ref, b_ref,
              y2l_ref, y2r_ref, inv_ref):
    s = p_ref[0] + p_ref[1]
    cnt = jnp.sum(cnt_ref[...], axis=0)
    inv = 1.0 / jnp.clip(cnt, 1.0, None)
    h1 = jnp.maximum(s * inv[:, None] + yr_ref[...], 0.0)
    y2l_ref[...] = jnp.dot(h1, wl_ref[...], preferred_element_type=jnp.float32)
    y2r_ref[...] = (jnp.dot(h1, wr_ref[...], preferred_element_type=jnp.float32)
                    + b_ref[...])
    inv_ref[...] = inv[:, None]


def _tc3_body(p_ref, inv_ref, yr_ref, wf_ref, bf_ref, out_ref):
    s = p_ref[0] + p_ref[1]
    h2 = jnp.maximum(s * inv_ref[...] + yr_ref[...], 0.0)
    out_ref[...] = (jnp.dot(h2, wf_ref[...], preferred_element_type=jnp.float32)
                    + bf_ref[...])


def kernel(x, edge_index, tau, W1l, W1r, b1, W2l, W2r, b2, Wf, bf):
    n, d = x.shape
    h = W1l.shape[1]
    e = edge_index.shape[1]

    ept = -(-e // NW)
    nch = -(-ept // CHUNK)
    ep = NW * nch * CHUNK
    n_pad = -(-(n + 1) // NS) * NS
    rpt = n_pad // NS

    src = jnp.concatenate([edge_index[0], jnp.zeros((ep - e,), jnp.int32)])
    dst = jnp.concatenate([edge_index[1], jnp.full((ep - e,), n, jnp.int32)])
    src3 = src.reshape(NW, nch, CHUNK)
    dst3 = dst.reshape(NW, nch, CHUNK)
    dstf = dst.reshape(NW, nch * CHUNK)
    zeros = jnp.zeros((rpt, h), jnp.float32)

    seg1 = _make_seg_kernel(n, n_pad, h, nch, True)
    seg2 = _make_seg_kernel(n, n_pad, h, nch, False)

    r = 400 if n % 400 == 0 else 8
    grid = n // r
    full = lambda i: (0, 0)
    row2 = pl.BlockSpec((r, h), lambda i: (i, 0))

    tc1 = pl.pallas_call(
        _tc1_body, grid=(grid,),
        in_specs=[pl.BlockSpec((r, d), lambda i: (i, 0)),
                  pl.BlockSpec((r, 1), lambda i: (i, 0)),
                  pl.BlockSpec((d + 1, h), full),
                  pl.BlockSpec((d + 1, h), full),
                  pl.BlockSpec((1, h), full)],
        out_specs=[row2, row2],
        out_shape=[jax.ShapeDtypeStruct((n, h), jnp.float32)] * 2,
    )
    tc2 = pl.pallas_call(
        _tc2_body, grid=(grid,),
        in_specs=[pl.BlockSpec((NC, r, h), lambda i: (0, i, 0)),
                  pl.BlockSpec((NW, r), lambda i: (0, i)),
                  row2,
                  pl.BlockSpec((h, h), full),
                  pl.BlockSpec((h, h), full),
                  pl.BlockSpec((1, h), full)],
        out_specs=[row2, row2, pl.BlockSpec((r, 1), lambda i: (i, 0))],
        out_shape=[jax.ShapeDtypeStruct((n, h), jnp.float32),
                   jax.ShapeDtypeStruct((n, h), jnp.float32),
                   jax.ShapeDtypeStruct((n, 1), jnp.float32)],
    )
    tc3 = pl.pallas_call(
        _tc3_body, grid=(grid,),
        in_specs=[pl.BlockSpec((NC, r, h), lambda i: (0, i, 0)),
                  pl.BlockSpec((r, 1), lambda i: (i, 0)),
                  row2,
                  pl.BlockSpec((h, 1), full),
                  pl.BlockSpec((1, 1), full)],
        out_specs=pl.BlockSpec((r, 1), lambda i: (i, 0)),
        out_shape=jax.ShapeDtypeStruct((n, 1), jnp.float32),
    )

    y1l, y1r = tc1(x, tau, W1l, W1r, b1.reshape(1, h))
    part1, cnt = seg1(y1l, src3, dst3, dstf, zeros)
    y2l, y2r, inv = tc2(part1, cnt, y1r, W2l, W2r, b2.reshape(1, h))
    (part2,) = seg2(y2l, src3, dst3, dstf, zeros)
    return tc3(part2, inv, y2r, Wf, bf.reshape(1, 1))


# speedup vs baseline: 4.7064x; 4.7064x over previous
import functools

import jax
import jax.numpy as jnp
from jax import lax
from jax.experimental import pallas as pl
from jax.experimental.pallas import tpu as pltpu
from jax.experimental.pallas import tpu_sc as plsc

NC = 2
NS = 16
NW = NC * NS
LANES = 16
CHUNK = 128


@functools.lru_cache(maxsize=None)
def _make_seg_kernel(n_nodes, n_pad, h, nch, with_counts):
    ept = nch * CHUNK
    rpt = n_pad // NS
    mesh = plsc.VectorSubcoreMesh(
        core_axis_name="c", subcore_axis_name="s",
        num_cores=NC, num_subcores=NS)

    out_type = [jax.ShapeDtypeStruct((NC, n_pad, h), jnp.float32)]
    if with_counts:
        out_type.append(jax.ShapeDtypeStruct((NC, n_pad), jnp.float32))

    scratch = [
        pltpu.VMEM((nch, CHUNK), jnp.int32),
        pltpu.VMEM((nch, CHUNK), jnp.int32),
        pltpu.VMEM((CHUNK, h), jnp.float32),
        pltpu.VMEM_SHARED((n_pad, h), jnp.float32),
        pltpu.SemaphoreType.DMA,
    ]
    if with_counts:
        scratch.insert(3, pltpu.VMEM((CHUNK,), jnp.float32))
        scratch.insert(4, pltpu.VMEM_SHARED((n_pad,), jnp.float32))

    @functools.partial(
        pl.kernel, out_type=tuple(out_type), mesh=mesh,
        scratch_types=tuple(scratch),
        compiler_params=pltpu.CompilerParams(needs_layout_passes=False))
    def k(*refs):
        if with_counts:
            (y_hbm, src_hbm, dst_hbm, zeros2_hbm, zeros1_hbm,
             part_out, cnt_out,
             src_v, dst_v, rows_v, ones_v, cnt_sh, acc_sh, sem) = refs
        else:
            (y_hbm, src_hbm, dst_hbm, zeros2_hbm, zeros1_hbm,
             part_out,
             src_v, dst_v, rows_v, acc_sh, sem) = refs
        cid = lax.axis_index("c")
        sid = lax.axis_index("s")
        wid = sid * NC + cid

        pltpu.sync_copy(src_hbm.at[wid], src_v)
        pltpu.sync_copy(dst_hbm.at[wid], dst_v)
        pltpu.sync_copy(zeros2_hbm, acc_sh.at[pl.ds(sid * rpt, rpt)])

        if with_counts:
            @pl.when(sid == 0)
            def _():
                pltpu.sync_copy(zeros1_hbm, cnt_sh)
            one16 = jnp.ones((LANES,), jnp.float32)

            def obody(i, c):
                ones_v[pl.ds(i * LANES, LANES)] = one16
                return c
            lax.fori_loop(0, CHUNK // LANES, obody, 0)

        plsc.subcore_barrier()

        def gbody(j, c):
            pltpu.async_copy(y_hbm.at[src_v.at[j]], rows_v, sem).wait()
            pltpu.sync_copy(rows_v, acc_sh.at[dst_v.at[j]], add=True)
            if with_counts:
                pltpu.sync_copy(ones_v, cnt_sh.at[dst_v.at[j]], add=True)
            return c
        lax.fori_loop(0, nch, gbody, 0)

        plsc.subcore_barrier()
        pltpu.sync_copy(acc_sh.at[pl.ds(sid * rpt, rpt)],
                        part_out.at[cid, pl.ds(sid * rpt, rpt)])
        if with_counts:
            @pl.when(sid == 0)
            def _():
                pltpu.sync_copy(cnt_sh, cnt_out.at[cid])

    return k


def _tc1_body(x_ref, tau_ref, wl_ref, wr_ref, b_ref, yl_ref, yr_ref):
    xb = x_ref[...]
    t = tau_ref[...]
    d = xb.shape[1]
    yl_ref[...] = (jnp.dot(xb, wl_ref[:d, :], preferred_element_type=jnp.float32)
                   + t * wl_ref[d:d + 1, :])
    yr_ref[...] = (jnp.dot(xb, wr_ref[:d, :], preferred_element_type=jnp.float32)
                   + t * wr_ref[d:d + 1, :] + b_ref[...])


def _tc2_body(p_ref, cnt_ref, yr_ref, wl_ref, wr_ref, b_ref,
              y2l_ref, y2r_ref, inv_ref):
    s = p_ref[0] + p_ref[1]
    cnt = jnp.sum(cnt_ref[...], axis=1)
    inv = 1.0 / jnp.clip(cnt, 1.0, None)
    h1 = jnp.maximum(s * inv[:, None] + yr_ref[...], 0.0)
    y2l_ref[...] = jnp.dot(h1, wl_ref[...], preferred_element_type=jnp.float32)
    y2r_ref[...] = (jnp.dot(h1, wr_ref[...], preferred_element_type=jnp.float32)
                    + b_ref[...])
    inv_ref[...] = inv[:, None]


def _tc3_body(p_ref, inv_ref, yr_ref, wf_ref, bf_ref, out_ref):
    s = p_ref[0] + p_ref[1]
    h2 = jnp.maximum(s * inv_ref[...] + yr_ref[...], 0.0)
    out_ref[...] = (jnp.dot(h2, wf_ref[...], preferred_element_type=jnp.float32)
                    + bf_ref[...])


def kernel(x, edge_index, tau, W1l, W1r, b1, W2l, W2r, b2, Wf, bf):
    n, d = x.shape
    h = W1l.shape[1]
    e = edge_index.shape[1]

    ept = -(-e // NW)
    nch = -(-ept // CHUNK)
    ep = NW * nch * CHUNK
    n_pad = -(-(n + 1) // (NS * 8)) * (NS * 8)
    rpt = n_pad // NS

    src = jnp.concatenate([edge_index[0], jnp.zeros((ep - e,), jnp.int32)])
    dst = jnp.concatenate([edge_index[1], jnp.full((ep - e,), n, jnp.int32)])
    src3 = src.reshape(NW, nch, CHUNK)
    dst3 = dst.reshape(NW, nch, CHUNK)
    zeros2 = jnp.zeros((rpt, h), jnp.float32)
    zeros1 = jnp.zeros((n_pad,), jnp.float32)

    seg1 = _make_seg_kernel(n, n_pad, h, nch, True)
    seg2 = _make_seg_kernel(n, n_pad, h, nch, False)

    r = 400 if n % 400 == 0 else 8
    grid = n // r
    full = lambda i: (0, 0)
    row2 = pl.BlockSpec((r, h), lambda i: (i, 0))

    tc1 = pl.pallas_call(
        _tc1_body, grid=(grid,),
        in_specs=[pl.BlockSpec((r, d), lambda i: (i, 0)),
                  pl.BlockSpec((r, 1), lambda i: (i, 0)),
                  pl.BlockSpec((d + 1, h), full),
                  pl.BlockSpec((d + 1, h), full),
                  pl.BlockSpec((1, h), full)],
        out_specs=[row2, row2],
        out_shape=[jax.ShapeDtypeStruct((n, h), jnp.float32)] * 2,
    )
    tc2 = pl.pallas_call(
        _tc2_body, grid=(grid,),
        in_specs=[pl.BlockSpec((NC, r, h), lambda i: (0, i, 0)),
                  pl.BlockSpec((r, NC), lambda i: (i, 0)),
                  row2,
                  pl.BlockSpec((h, h), full),
                  pl.BlockSpec((h, h), full),
                  pl.BlockSpec((1, h), full)],
        out_specs=[row2, row2, pl.BlockSpec((r, 1), lambda i: (i, 0))],
        out_shape=[jax.ShapeDtypeStruct((n, h), jnp.float32),
                   jax.ShapeDtypeStruct((n, h), jnp.float32),
                   jax.ShapeDtypeStruct((n, 1), jnp.float32)],
    )
    tc3 = pl.pallas_call(
        _tc3_body, grid=(grid,),
        in_specs=[pl.BlockSpec((NC, r, h), lambda i: (0, i, 0)),
                  pl.BlockSpec((r, 1), lambda i: (i, 0)),
                  row2,
                  pl.BlockSpec((h, 1), full),
                  pl.BlockSpec((1, 1), full)],
        out_specs=pl.BlockSpec((r, 1), lambda i: (i, 0)),
        out_shape=jax.ShapeDtypeStruct((n, 1), jnp.float32),
    )

    y1l, y1r = tc1(x, tau, W1l, W1r, b1.reshape(1, h))
    part1, cnt = seg1(y1l, src3, dst3, zeros2, zeros1)
    y2l, y2r, inv = tc2(part1, cnt.T, y1r, W2l, W2r, b2.reshape(1, h))
    (part2,) = seg2(y2l, src3, dst3, zeros2, zeros1)
    return tc3(part2, inv, y2r, Wf, bf.reshape(1, 1))


# speedup vs baseline: 6.7739x; 1.4393x over previous
import functools

import jax
import jax.numpy as jnp
from jax import lax
from jax.experimental import pallas as pl
from jax.experimental.pallas import tpu as pltpu
from jax.experimental.pallas import tpu_sc as plsc

NC = 2
NS = 16
LANES = 16
CHUNK = 128
RING = 2


@functools.lru_cache(maxsize=None)
def _make_seg_kernel(n_nodes, n_pad, hh, nch, with_counts):
    rpt = n_pad // NS
    mesh = plsc.VectorSubcoreMesh(
        core_axis_name="c", subcore_axis_name="s",
        num_cores=NC, num_subcores=NS)

    out_type = [jax.ShapeDtypeStruct((NC, n_pad, hh), jnp.float32)]
    if with_counts:
        out_type.append(jax.ShapeDtypeStruct((NC, n_pad), jnp.float32))

    scratch = [
        pltpu.VMEM((nch, CHUNK), jnp.int32),
        pltpu.VMEM((nch, CHUNK), jnp.int32),
        pltpu.VMEM((RING, CHUNK, hh), jnp.float32),
        pltpu.VMEM_SHARED((n_pad, hh), jnp.float32),
        pltpu.SemaphoreType.DMA,
    ]
    if with_counts:
        scratch.insert(3, pltpu.VMEM((CHUNK,), jnp.float32))
        scratch.insert(4, pltpu.VMEM_SHARED((n_pad,), jnp.float32))

    @functools.partial(
        pl.kernel, out_type=tuple(out_type), mesh=mesh,
        scratch_types=tuple(scratch),
        compiler_params=pltpu.CompilerParams(needs_layout_passes=False,
                                             use_tc_tiling_on_sc=False))
    def k(*refs):
        if with_counts:
            (ya_hbm, yb_hbm, src_hbm, dst_hbm, zeros2_hbm, zeros1_hbm,
             part_out, cnt_out,
             src_v, dst_v, rows_v, ones_v, cnt_sh, acc_sh, sem) = refs
        else:
            (ya_hbm, yb_hbm, src_hbm, dst_hbm, zeros2_hbm, zeros1_hbm,
             part_out,
             src_v, dst_v, rows_v, acc_sh, sem) = refs
        cid = lax.axis_index("c")
        sid = lax.axis_index("s")

        pltpu.sync_copy(src_hbm.at[sid], src_v)
        pltpu.sync_copy(dst_hbm.at[sid], dst_v)
        pltpu.sync_copy(zeros2_hbm, acc_sh.at[pl.ds(sid * rpt, rpt)])

        if with_counts:
            @pl.when(sid == 0)
            def _():
                pltpu.sync_copy(zeros1_hbm, cnt_sh)
            one16 = jnp.ones((LANES,), jnp.float32)

            def obody(i, c):
                ones_v[pl.ds(i * LANES, LANES)] = one16
                return c
            lax.fori_loop(0, CHUNK // LANES, obody, 0)

        plsc.subcore_barrier()

        def start_gather(j, b):
            @pl.when(cid == 0)
            def _():
                pltpu.async_copy(ya_hbm.at[src_v.at[j]], rows_v.at[b], sem)

            @pl.when(cid == 1)
            def _():
                pltpu.async_copy(yb_hbm.at[src_v.at[j]], rows_v.at[b], sem)

        def drain(j, b):
            pltpu.make_async_copy(ya_hbm.at[src_v.at[0]], rows_v.at[b],
                                  sem).wait()
            pltpu.sync_copy(rows_v.at[b], acc_sh.at[dst_v.at[j]], add=True)
            if with_counts:
                @pl.when(lax.rem(j, 2) == cid)
                def _():
                    pltpu.sync_copy(ones_v, cnt_sh.at[dst_v.at[j]], add=True)

        for b in range(RING):
            start_gather(b, b)

        @pl.loop(0, nch - RING, step=RING)
        def _(j0):
            for b in range(RING):
                drain(j0 + b, b)
                start_gather(j0 + b + RING, b)

        for b in range(RING):
            drain(nch - RING + b, b)

        plsc.subcore_barrier()
        pltpu.sync_copy(acc_sh.at[pl.ds(sid * rpt, rpt)],
                        part_out.at[cid, pl.ds(sid * rpt, rpt)])
        if with_counts:
            @pl.when(sid == 0)
            def _():
                pltpu.sync_copy(cnt_sh, cnt_out.at[cid])

    return k


def _tc1_body(x_ref, tau_ref, wl_ref, wr_ref, b_ref, ya_ref, yb_ref, yr_ref):
    xb = x_ref[...]
    t = tau_ref[...]
    d = xb.shape[1]
    hh = ya_ref.shape[1]
    yl = (jnp.dot(xb, wl_ref[:d, :], preferred_element_type=jnp.float32)
          + t * wl_ref[d:d + 1, :])
    ya_ref[...] = yl[:, :hh]
    yb_ref[...] = yl[:, hh:]
    yr_ref[...] = (jnp.dot(xb, wr_ref[:d, :], preferred_element_type=jnp.float32)
                   + t * wr_ref[d:d + 1, :] + b_ref[...])


def _tc2_body(p_ref, cnt_ref, yr_ref, wl_ref, wr_ref, b_ref,
              y2a_ref, y2b_ref, y2r_ref, inv_ref):
    s = jnp.concatenate([p_ref[0], p_ref[1]], axis=1)
    hh = y2a_ref.shape[1]
    cnt = cnt_ref[0, :, 0] + cnt_ref[1, :, 0]
    inv = 1.0 / jnp.clip(cnt, 1.0, None)
    h1 = jnp.maximum(s * inv[:, None] + yr_ref[...], 0.0)
    y2l = jnp.dot(h1, wl_ref[...], preferred_element_type=jnp.float32)
    y2a_ref[...] = y2l[:, :hh]
    y2b_ref[...] = y2l[:, hh:]
    y2r_ref[...] = (jnp.dot(h1, wr_ref[...], preferred_element_type=jnp.float32)
                    + b_ref[...])
    inv_ref[...] = inv[:, None]


def _tc3_body(p_ref, inv_ref, yr_ref, wf_ref, bf_ref, out_ref):
    s = jnp.concatenate([p_ref[0], p_ref[1]], axis=1)
    h2 = jnp.maximum(s * inv_ref[...] + yr_ref[...], 0.0)
    out_ref[...] = (jnp.dot(h2, wf_ref[...], preferred_element_type=jnp.float32)
                    + bf_ref[...])


def kernel(x, edge_index, tau, W1l, W1r, b1, W2l, W2r, b2, Wf, bf):
    n, d = x.shape
    h = W1l.shape[1]
    hh = h // 2
    e = edge_index.shape[1]

    ept = -(-e // NS)
    nch = -(-ept // CHUNK)
    nch = -(-nch // RING) * RING
    ep = NS * nch * CHUNK
    n_pad = -(-(n + 1) // (NS * 8)) * (NS * 8)
    rpt = n_pad // NS

    src = jnp.concatenate([edge_index[0], jnp.zeros((ep - e,), jnp.int32)])
    dst = jnp.concatenate([edge_index[1], jnp.full((ep - e,), n, jnp.int32)])
    src3 = src.reshape(NS, nch, CHUNK)
    dst3 = dst.reshape(NS, nch, CHUNK)
    zeros2 = jnp.zeros((rpt, hh), jnp.float32)
    zeros1 = jnp.zeros((n_pad,), jnp.float32)

    seg1 = _make_seg_kernel(n, n_pad, hh, nch, True)
    seg2 = _make_seg_kernel(n, n_pad, hh, nch, False)

    r = 400 if n % 400 == 0 else 8
    grid = n // r
    full = lambda i: (0, 0)
    row2 = pl.BlockSpec((r, h), lambda i: (i, 0))
    rowh = pl.BlockSpec((r, hh), lambda i: (i, 0))
    col1 = pl.BlockSpec((r, 1), lambda i: (i, 0))
    part_spec = pl.BlockSpec((NC, r, hh), lambda i: (0, i, 0))

    tc1 = pl.pallas_call(
        _tc1_body, grid=(grid,),
        in_specs=[pl.BlockSpec((r, d), lambda i: (i, 0)),
                  col1,
                  pl.BlockSpec((d + 1, h), full),
                  pl.BlockSpec((d + 1, h), full),
                  pl.BlockSpec((1, h), full)],
        out_specs=[rowh, rowh, row2],
        out_shape=[jax.ShapeDtypeStruct((n, hh), jnp.float32),
                   jax.ShapeDtypeStruct((n, hh), jnp.float32),
                   jax.ShapeDtypeStruct((n, h), jnp.float32)],
    )
    tc2 = pl.pallas_call(
        _tc2_body, grid=(grid,),
        in_specs=[part_spec,
                  pl.BlockSpec((NC, r, 1), lambda i: (0, i, 0)),
                  row2,
                  pl.BlockSpec((h, h), full),
                  pl.BlockSpec((h, h), full),
                  pl.BlockSpec((1, h), full)],
        out_specs=[rowh, rowh, row2, col1],
        out_shape=[jax.ShapeDtypeStruct((n, hh), jnp.float32),
                   jax.ShapeDtypeStruct((n, hh), jnp.float32),
                   jax.ShapeDtypeStruct((n, h), jnp.float32),
                   jax.ShapeDtypeStruct((n, 1), jnp.float32)],
    )
    tc3 = pl.pallas_call(
        _tc3_body, grid=(grid,),
        in_specs=[part_spec,
                  col1,
                  row2,
                  pl.BlockSpec((h, 1), full),
                  pl.BlockSpec((1, 1), full)],
        out_specs=col1,
        out_shape=jax.ShapeDtypeStruct((n, 1), jnp.float32),
    )

    y1a, y1b, y1r = tc1(x, tau, W1l, W1r, b1.reshape(1, h))
    part1, cnt = seg1(y1a, y1b, src3, dst3, zeros2, zeros1)
    y2a, y2b, y2r, inv = tc2(part1, cnt.reshape(NC, n_pad, 1), y1r,
                             W2l, W2r, b2.reshape(1, h))
    (part2,) = seg2(y2a, y2b, src3, dst3, zeros2, zeros1)
    return tc3(part2, inv, y2r, Wf, bf.reshape(1, 1))


# speedup vs baseline: 8.2403x; 1.2165x over previous
import functools

import jax
import jax.numpy as jnp
from jax import lax
from jax.experimental import pallas as pl
from jax.experimental.pallas import tpu as pltpu
from jax.experimental.pallas import tpu_sc as plsc

NC = 2
NS = 16
LANES = 16
CHUNK = 128
RING = 2


@functools.lru_cache(maxsize=None)
def _make_seg_kernel(n_nodes, n_pad, hh, nch, with_counts):
    rpt = n_pad // NS
    mesh = plsc.VectorSubcoreMesh(
        core_axis_name="c", subcore_axis_name="s",
        num_cores=NC, num_subcores=NS)

    out_type = [jax.ShapeDtypeStruct((NC, n_pad, hh), jnp.float32)]
    if with_counts:
        out_type.append(jax.ShapeDtypeStruct((NC, n_pad), jnp.float32))

    scratch = [
        pltpu.VMEM((4, CHUNK), jnp.int32),
        pltpu.VMEM((4, CHUNK), jnp.int32),
        pltpu.VMEM((RING, CHUNK, hh), jnp.float32),
        pltpu.VMEM_SHARED((n_nodes, hh), jnp.float32),
        pltpu.VMEM_SHARED((n_pad, hh), jnp.float32),
        pltpu.SemaphoreType.DMA,
        pltpu.SemaphoreType.DMA,
    ]
    if with_counts:
        scratch.insert(3, pltpu.VMEM((CHUNK,), jnp.float32))
        scratch.insert(4, pltpu.VMEM_SHARED((n_pad,), jnp.float32))

    @functools.partial(
        pl.kernel, out_type=tuple(out_type), mesh=mesh,
        scratch_types=tuple(scratch),
        compiler_params=pltpu.CompilerParams(needs_layout_passes=False,
                                             use_tc_tiling_on_sc=False))
    def k(*refs):
        if with_counts:
            (ya_hbm, yb_hbm, src_hbm, dst_hbm, zeros2_hbm, zeros1_hbm,
             part_out, cnt_out,
             sring, dring, rows_v, ones_v, cnt_sh, y_sh, acc_sh,
             sem_g, sem_i) = refs
        else:
            (ya_hbm, yb_hbm, src_hbm, dst_hbm, zeros2_hbm, zeros1_hbm,
             part_out,
             sring, dring, rows_v, y_sh, acc_sh, sem_g, sem_i) = refs
        cid = lax.axis_index("c")
        sid = lax.axis_index("s")

        @pl.when(sid == 0)
        def _():
            @pl.when(cid == 0)
            def _():
                pltpu.sync_copy(ya_hbm, y_sh)

            @pl.when(cid == 1)
            def _():
                pltpu.sync_copy(yb_hbm, y_sh)

        pltpu.sync_copy(zeros2_hbm, acc_sh.at[pl.ds(sid * rpt, rpt)])

        if with_counts:
            @pl.when(sid == 1)
            def _():
                pltpu.sync_copy(zeros1_hbm, cnt_sh)
            one16 = jnp.ones((LANES,), jnp.float32)

            def obody(i, c):
                ones_v[pl.ds(i * LANES, LANES)] = one16
                return c
            lax.fori_loop(0, CHUNK // LANES, obody, 0)

        def load_pair(j, slot):
            pltpu.async_copy(src_hbm.at[sid, j], sring.at[slot], sem_i)
            pltpu.async_copy(dst_hbm.at[sid, j], dring.at[slot], sem_i)

        def wait_pair(slot):
            for ring in (sring, dring):
                pltpu.make_async_copy(src_hbm.at[sid, 0], ring.at[slot],
                                      sem_i).wait()

        def start_gather(slot, b):
            pltpu.async_copy(y_sh.at[sring.at[slot]], rows_v.at[b], sem_g)

        def drain(slot, b, count_core):
            pltpu.make_async_copy(ya_hbm.at[sring.at[0]], rows_v.at[b],
                                  sem_g).wait()
            pltpu.sync_copy(rows_v.at[b], acc_sh.at[dring.at[slot]], add=True)
            if with_counts:
                @pl.when(cid == count_core)
                def _():
                    pltpu.sync_copy(ones_v, cnt_sh.at[dring.at[slot]],
                                    add=True)

        for j in range(4):
            load_pair(j, j)

        plsc.subcore_barrier()

        for b in range(RING):
            wait_pair(b)
            start_gather(b, b)

        @pl.loop(0, nch - 4, step=4)
        def _(j0):
            for b in range(4):
                drain(b, b % RING, b % NC)
                load_pair(j0 + b + 4, b)
                wait_pair((b + RING) % 4)
                start_gather((b + RING) % 4, b % RING)

        for b in range(4):
            slot, rb = b, b % RING
            drain(slot, rb, b % NC)
            if b < RING:
                wait_pair((b + RING) % 4)
                start_gather((b + RING) % 4, rb)

        plsc.subcore_barrier()
        pltpu.sync_copy(acc_sh.at[pl.ds(sid * rpt, rpt)],
                        part_out.at[cid, pl.ds(sid * rpt, rpt)])
        if with_counts:
            @pl.when(sid == 1)
            def _():
                pltpu.sync_copy(cnt_sh, cnt_out.at[cid])

    return k


def _tc1_body(x_ref, tau_ref, wl_ref, wr_ref, b_ref, ya_ref, yb_ref, yr_ref):
    xb = x_ref[...]
    t = tau_ref[...]
    d = xb.shape[1]
    hh = ya_ref.shape[1]
    yl = (jnp.dot(xb, wl_ref[:d, :], preferred_element_type=jnp.float32)
          + t * wl_ref[d:d + 1, :])
    ya_ref[...] = yl[:, :hh]
    yb_ref[...] = yl[:, hh:]
    yr_ref[...] = (jnp.dot(xb, wr_ref[:d, :], preferred_element_type=jnp.float32)
                   + t * wr_ref[d:d + 1, :] + b_ref[...])


def _tc2_body(p_ref, cnt_ref, yr_ref, wl_ref, wr_ref, b_ref,
              y2a_ref, y2b_ref, y2r_ref, inv_ref):
    s = jnp.concatenate([p_ref[0], p_ref[1]], axis=1)
    hh = y2a_ref.shape[1]
    cnt = cnt_ref[0, :, 0] + cnt_ref[1, :, 0]
    inv = 1.0 / jnp.clip(cnt, 1.0, None)
    h1 = jnp.maximum(s * inv[:, None] + yr_ref[...], 0.0)
    y2l = jnp.dot(h1, wl_ref[...], preferred_element_type=jnp.float32)
    y2a_ref[...] = y2l[:, :hh]
    y2b_ref[...] = y2l[:, hh:]
    y2r_ref[...] = (jnp.dot(h1, wr_ref[...], preferred_element_type=jnp.float32)
                    + b_ref[...])
    inv_ref[...] = inv[:, None]


def _tc3_body(p_ref, inv_ref, yr_ref, wf_ref, bf_ref, out_ref):
    s = jnp.concatenate([p_ref[0], p_ref[1]], axis=1)
    h2 = jnp.maximum(s * inv_ref[...] + yr_ref[...], 0.0)
    out_ref[...] = (jnp.dot(h2, wf_ref[...], preferred_element_type=jnp.float32)
                    + bf_ref[...])


def kernel(x, edge_index, tau, W1l, W1r, b1, W2l, W2r, b2, Wf, bf):
    n, d = x.shape
    h = W1l.shape[1]
    hh = h // 2
    e = edge_index.shape[1]

    ept = -(-e // NS)
    nch = -(-ept // CHUNK)
    nch = -(-nch // 4) * 4
    ep = NS * nch * CHUNK
    n_pad = -(-(n + 1) // (NS * 8)) * (NS * 8)
    rpt = n_pad // NS

    src = jnp.concatenate([edge_index[0], jnp.zeros((ep - e,), jnp.int32)])
    dst = jnp.concatenate([edge_index[1], jnp.full((ep - e,), n, jnp.int32)])
    src3 = src.reshape(NS, nch, CHUNK)
    dst3 = dst.reshape(NS, nch, CHUNK)
    zeros2 = jnp.zeros((rpt, hh), jnp.float32)
    zeros1 = jnp.zeros((n_pad,), jnp.float32)

    seg1 = _make_seg_kernel(n, n_pad, hh, nch, True)
    seg2 = _make_seg_kernel(n, n_pad, hh, nch, False)

    r = 400 if n % 400 == 0 else 8
    grid = n // r
    full = lambda i: (0, 0)
    row2 = pl.BlockSpec((r, h), lambda i: (i, 0))
    rowh = pl.BlockSpec((r, hh), lambda i: (i, 0))
    col1 = pl.BlockSpec((r, 1), lambda i: (i, 0))
    part_spec = pl.BlockSpec((NC, r, hh), lambda i: (0, i, 0))

    tc1 = pl.pallas_call(
        _tc1_body, grid=(grid,),
        in_specs=[pl.BlockSpec((r, d), lambda i: (i, 0)),
                  col1,
                  pl.BlockSpec((d + 1, h), full),
                  pl.BlockSpec((d + 1, h), full),
                  pl.BlockSpec((1, h), full)],
        out_specs=[rowh, rowh, row2],
        out_shape=[jax.ShapeDtypeStruct((n, hh), jnp.float32),
                   jax.ShapeDtypeStruct((n, hh), jnp.float32),
                   jax.ShapeDtypeStruct((n, h), jnp.float32)],
    )
    tc2 = pl.pallas_call(
        _tc2_body, grid=(grid,),
        in_specs=[part_spec,
                  pl.BlockSpec((NC, r, 1), lambda i: (0, i, 0)),
                  row2,
                  pl.BlockSpec((h, h), full),
                  pl.BlockSpec((h, h), full),
                  pl.BlockSpec((1, h), full)],
        out_specs=[rowh, rowh, row2, col1],
        out_shape=[jax.ShapeDtypeStruct((n, hh), jnp.float32),
                   jax.ShapeDtypeStruct((n, hh), jnp.float32),
                   jax.ShapeDtypeStruct((n, h), jnp.float32),
                   jax.ShapeDtypeStruct((n, 1), jnp.float32)],
    )
    tc3 = pl.pallas_call(
        _tc3_body, grid=(grid,),
        in_specs=[part_spec,
                  col1,
                  row2,
                  pl.BlockSpec((h, 1), full),
                  pl.BlockSpec((1, 1), full)],
        out_specs=col1,
        out_shape=jax.ShapeDtypeStruct((n, 1), jnp.float32),
    )

    y1a, y1b, y1r = tc1(x, tau, W1l, W1r, b1.reshape(1, h))
    part1, cnt = seg1(y1a, y1b, src3, dst3, zeros2, zeros1)
    y2a, y2b, y2r, inv = tc2(part1, cnt.reshape(NC, n_pad, 1), y1r,
                             W2l, W2r, b2.reshape(1, h))
    (part2,) = seg2(y2a, y2b, src3, dst3, zeros2, zeros1)
    return tc3(part2, inv, y2r, Wf, bf.reshape(1, 1))


# speedup vs baseline: 8.8032x; 1.0683x over previous
import functools

import jax
import jax.numpy as jnp
from jax import lax
from jax.experimental import pallas as pl
from jax.experimental.pallas import tpu as pltpu
from jax.experimental.pallas import tpu_sc as plsc

NC = 2
NS = 16
LANES = 16
CHUNK = 128
RING = 2


@functools.lru_cache(maxsize=None)
def _make_seg_kernel(n_nodes, n_pad, hh, nch, with_counts):
    rpt = n_pad // NS
    mesh = plsc.VectorSubcoreMesh(
        core_axis_name="c", subcore_axis_name="s",
        num_cores=NC, num_subcores=NS)

    out_type = [jax.ShapeDtypeStruct((NC, n_pad, hh), jnp.float32)]
    if with_counts:
        out_type.append(jax.ShapeDtypeStruct((NC, n_pad), jnp.float32))

    scratch = [
        pltpu.VMEM((4, CHUNK), jnp.int32),
        pltpu.VMEM((4, CHUNK), jnp.int32),
        pltpu.VMEM((RING, CHUNK, hh), jnp.float32),
        pltpu.VMEM_SHARED((n_nodes, hh), jnp.float32),
        pltpu.VMEM_SHARED((n_pad, hh), jnp.float32),
        pltpu.SemaphoreType.DMA,
        pltpu.SemaphoreType.DMA,
    ]
    if with_counts:
        scratch.insert(3, pltpu.VMEM((CHUNK,), jnp.float32))
        scratch.insert(4, pltpu.VMEM_SHARED((n_pad,), jnp.float32))

    @functools.partial(
        pl.kernel, out_type=tuple(out_type), mesh=mesh,
        scratch_types=tuple(scratch),
        compiler_params=pltpu.CompilerParams(needs_layout_passes=False,
                                             use_tc_tiling_on_sc=False))
    def k(*refs):
        if with_counts:
            (ya_hbm, yb_hbm, src_hbm, dst_hbm, zeros2_hbm, zeros1_hbm,
             part_out, cnt_out,
             sring, dring, rows_v, ones_v, cnt_sh, y_sh, acc_sh,
             sem_g, sem_i) = refs
        else:
            (ya_hbm, yb_hbm, src_hbm, dst_hbm, zeros2_hbm, zeros1_hbm,
             part_out,
             sring, dring, rows_v, y_sh, acc_sh, sem_g, sem_i) = refs
        cid = lax.axis_index("c")
        sid = lax.axis_index("s")

        @pl.when(sid == 0)
        def _():
            @pl.when(cid == 0)
            def _():
                pltpu.sync_copy(ya_hbm, y_sh)

            @pl.when(cid == 1)
            def _():
                pltpu.sync_copy(yb_hbm, y_sh)

        pltpu.sync_copy(zeros2_hbm, acc_sh.at[pl.ds(sid * rpt, rpt)])

        if with_counts:
            @pl.when(sid == 1)
            def _():
                pltpu.sync_copy(zeros1_hbm, cnt_sh)
            one16 = jnp.ones((LANES,), jnp.float32)

            def obody(i, c):
                ones_v[pl.ds(i * LANES, LANES)] = one16
                return c
            lax.fori_loop(0, CHUNK // LANES, obody, 0)

        def load_pair(j, slot):
            pltpu.async_copy(src_hbm.at[sid, j], sring.at[slot], sem_i)
            pltpu.async_copy(dst_hbm.at[sid, j], dring.at[slot], sem_i)

        def wait_pair(slot):
            for ring in (sring, dring):
                pltpu.make_async_copy(src_hbm.at[sid, 0], ring.at[slot],
                                      sem_i).wait()

        def start_gather(slot, b):
            pltpu.async_copy(y_sh.at[sring.at[slot]], rows_v.at[b], sem_g)

        def drain(slot, b, count_core):
            pltpu.make_async_copy(ya_hbm.at[sring.at[0]], rows_v.at[b],
                                  sem_g).wait()
            pltpu.sync_copy(rows_v.at[b], acc_sh.at[dring.at[slot]], add=True)
            if with_counts:
                @pl.when(cid == count_core)
                def _():
                    pltpu.sync_copy(ones_v, cnt_sh.at[dring.at[slot]],
                                    add=True)

        for j in range(4):
            load_pair(j, j)

        plsc.subcore_barrier()

        for b in range(RING):
            wait_pair(b)
            start_gather(b, b)

        @pl.loop(0, nch - 4, step=4)
        def _(j0):
            for b in range(4):
                drain(b, b % RING, b % NC)
                load_pair(j0 + b + 4, b)
                wait_pair((b + RING) % 4)
                start_gather((b + RING) % 4, b % RING)

        for b in range(4):
            slot, rb = b, b % RING
            drain(slot, rb, b % NC)
            if b < RING:
                wait_pair((b + RING) % 4)
                start_gather((b + RING) % 4, rb)

        plsc.subcore_barrier()
        pltpu.sync_copy(acc_sh.at[pl.ds(sid * rpt, rpt)],
                        part_out.at[cid, pl.ds(sid * rpt, rpt)])
        if with_counts:
            @pl.when(sid == 1)
            def _():
                pltpu.sync_copy(cnt_sh, cnt_out.at[cid])

    return k


def _tc1_body(x_ref, tau_ref, wl_ref, wr_ref, b_ref, ya_ref, yb_ref, yr_ref):
    xb = x_ref[...]
    t = tau_ref[...]
    d = xb.shape[1]
    hh = ya_ref.shape[1]
    yl = (jnp.dot(xb, wl_ref[:d, :], preferred_element_type=jnp.float32)
          + t * wl_ref[d:d + 1, :])
    ya_ref[...] = yl[:, :hh]
    yb_ref[...] = yl[:, hh:]
    yr_ref[...] = (jnp.dot(xb, wr_ref[:d, :], preferred_element_type=jnp.float32)
                   + t * wr_ref[d:d + 1, :] + b_ref[...])


def _tc2_body(p_ref, cnt_ref, yr_ref, wl_ref, wr_ref, b_ref,
              y2a_ref, y2b_ref, y2r_ref, inv_ref):
    s = jnp.concatenate([p_ref[0], p_ref[1]], axis=1)
    hh = y2a_ref.shape[1]
    cnt = cnt_ref[0, :, 0] + cnt_ref[1, :, 0]
    inv = 1.0 / jnp.clip(cnt, 1.0, None)
    h1 = jnp.maximum(s * inv[:, None] + yr_ref[...], 0.0)
    y2l = jnp.dot(h1, wl_ref[...], preferred_element_type=jnp.float32)
    y2a_ref[...] = y2l[:, :hh]
    y2b_ref[...] = y2l[:, hh:]
    y2r_ref[...] = (jnp.dot(h1, wr_ref[...], preferred_element_type=jnp.float32)
                    + b_ref[...])
    inv_ref[...] = inv[:, None]


def _tc3_body(p_ref, inv_ref, yr_ref, wf_ref, bf_ref, out_ref):
    s = jnp.concatenate([p_ref[0], p_ref[1]], axis=1)
    h2 = jnp.maximum(s * inv_ref[...] + yr_ref[...], 0.0)
    out_ref[...] = (jnp.dot(h2, wf_ref[...], preferred_element_type=jnp.float32)
                    + bf_ref[...])


def kernel(x, edge_index, tau, W1l, W1r, b1, W2l, W2r, b2, Wf, bf):
    n, d = x.shape
    h = W1l.shape[1]
    hh = h // 2
    e = edge_index.shape[1]

    ept = -(-e // NS)
    nch = -(-ept // CHUNK)
    nch = -(-nch // 4) * 4
    ep = NS * nch * CHUNK
    n_pad = -(-(n + 1) // (NS * 8)) * (NS * 8)
    rpt = n_pad // NS

    src = jnp.concatenate([edge_index[0], jnp.zeros((ep - e,), jnp.int32)])
    dst = jnp.concatenate([edge_index[1], jnp.full((ep - e,), n, jnp.int32)])
    src3 = src.reshape(NS, nch, CHUNK)
    dst3 = dst.reshape(NS, nch, CHUNK)
    zeros2 = jnp.zeros((rpt, hh), jnp.float32)
    zeros1 = jnp.zeros((n_pad,), jnp.float32)

    seg1 = _make_seg_kernel(n, n_pad, hh, nch, True)
    seg2 = _make_seg_kernel(n, n_pad, hh, nch, False)

    r = n if n % 8 == 0 else 8 * (-(-n // 8))
    grid = n // r if n % 8 == 0 else 1
    full = lambda i: (0, 0)
    row2 = pl.BlockSpec((r, h), lambda i: (i, 0))
    rowh = pl.BlockSpec((r, hh), lambda i: (i, 0))
    col1 = pl.BlockSpec((r, 1), lambda i: (i, 0))
    part_spec = pl.BlockSpec((NC, r, hh), lambda i: (0, i, 0))

    tc1 = pl.pallas_call(
        _tc1_body, grid=(grid,),
        in_specs=[pl.BlockSpec((r, d), lambda i: (i, 0)),
                  col1,
                  pl.BlockSpec((d + 1, h), full),
                  pl.BlockSpec((d + 1, h), full),
                  pl.BlockSpec((1, h), full)],
        out_specs=[rowh, rowh, row2],
        out_shape=[jax.ShapeDtypeStruct((n, hh), jnp.float32),
                   jax.ShapeDtypeStruct((n, hh), jnp.float32),
                   jax.ShapeDtypeStruct((n, h), jnp.float32)],
    )
    tc2 = pl.pallas_call(
        _tc2_body, grid=(grid,),
        in_specs=[part_spec,
                  pl.BlockSpec((NC, r, 1), lambda i: (0, i, 0)),
                  row2,
                  pl.BlockSpec((h, h), full),
                  pl.BlockSpec((h, h), full),
                  pl.BlockSpec((1, h), full)],
        out_specs=[rowh, rowh, row2, col1],
        out_shape=[jax.ShapeDtypeStruct((n, hh), jnp.float32),
                   jax.ShapeDtypeStruct((n, hh), jnp.float32),
                   jax.ShapeDtypeStruct((n, h), jnp.float32),
                   jax.ShapeDtypeStruct((n, 1), jnp.float32)],
    )
    tc3 = pl.pallas_call(
        _tc3_body, grid=(grid,),
        in_specs=[part_spec,
                  col1,
                  row2,
                  pl.BlockSpec((h, 1), full),
                  pl.BlockSpec((1, 1), full)],
        out_specs=col1,
        out_shape=jax.ShapeDtypeStruct((n, 1), jnp.float32),
    )

    y1a, y1b, y1r = tc1(x, tau, W1l, W1r, b1.reshape(1, h))
    part1, cnt = seg1(y1a, y1b, src3, dst3, zeros2, zeros1)
    y2a, y2b, y2r, inv = tc2(part1, cnt.reshape(NC, n_pad, 1), y1r,
                             W2l, W2r, b2.reshape(1, h))
    (part2,) = seg2(y2a, y2b, src3, dst3, zeros2, zeros1)
    return tc3(part2, inv, y2r, Wf, bf.reshape(1, 1))


# speedup vs baseline: 9.4712x; 1.0759x over previous
import functools

import jax
import jax.numpy as jnp
from jax import lax
from jax.experimental import pallas as pl
from jax.experimental.pallas import tpu as pltpu
from jax.experimental.pallas import tpu_sc as plsc

NC = 2
NS = 16
LANES = 16
CHUNK = 128
RING = 4
IRING = 8


@functools.lru_cache(maxsize=None)
def _make_seg_kernel(n_nodes, n_pad, hh, nch, with_counts):
    rpt = n_pad // NS
    mesh = plsc.VectorSubcoreMesh(
        core_axis_name="c", subcore_axis_name="s",
        num_cores=NC, num_subcores=NS)

    out_type = [jax.ShapeDtypeStruct((NC, n_pad, hh), jnp.float32)]
    if with_counts:
        out_type.append(jax.ShapeDtypeStruct((NC, n_pad), jnp.float32))

    scratch = [
        pltpu.VMEM((IRING, CHUNK), jnp.int32),
        pltpu.VMEM((IRING, CHUNK), jnp.int32),
        pltpu.VMEM((RING, CHUNK, hh), jnp.float32),
        pltpu.VMEM_SHARED((n_nodes, hh), jnp.float32),
        pltpu.VMEM_SHARED((n_pad, hh), jnp.float32),
        pltpu.SemaphoreType.DMA,
        pltpu.SemaphoreType.DMA,
        pltpu.SemaphoreType.DMA,
    ]
    if with_counts:
        scratch.insert(3, pltpu.VMEM((CHUNK,), jnp.float32))
        scratch.insert(4, pltpu.VMEM_SHARED((n_pad,), jnp.float32))

    @functools.partial(
        pl.kernel, out_type=tuple(out_type), mesh=mesh,
        scratch_types=tuple(scratch),
        compiler_params=pltpu.CompilerParams(needs_layout_passes=False,
                                             use_tc_tiling_on_sc=False))
    def k(*refs):
        if with_counts:
            (ya_hbm, yb_hbm, src_hbm, dst_hbm, zeros2_hbm, zeros1_hbm,
             part_out, cnt_out,
             sring, dring, rows_v, ones_v, cnt_sh, y_sh, acc_sh,
             sem_g, sem_i, sem_s) = refs
        else:
            (ya_hbm, yb_hbm, src_hbm, dst_hbm, zeros2_hbm, zeros1_hbm,
             part_out,
             sring, dring, rows_v, y_sh, acc_sh, sem_g, sem_i, sem_s) = refs
        cid = lax.axis_index("c")
        sid = lax.axis_index("s")

        @pl.when(sid == 0)
        def _():
            @pl.when(cid == 0)
            def _():
                pltpu.sync_copy(ya_hbm, y_sh)

            @pl.when(cid == 1)
            def _():
                pltpu.sync_copy(yb_hbm, y_sh)

        pltpu.sync_copy(zeros2_hbm, acc_sh.at[pl.ds(sid * rpt, rpt)])

        if with_counts:
            @pl.when(sid == 1)
            def _():
                pltpu.sync_copy(zeros1_hbm, cnt_sh)
            one16 = jnp.ones((LANES,), jnp.float32)

            def obody(i, c):
                ones_v[pl.ds(i * LANES, LANES)] = one16
                return c
            lax.fori_loop(0, CHUNK // LANES, obody, 0)

        def load_pair(j, slot):
            pltpu.async_copy(src_hbm.at[sid, j], sring.at[slot], sem_i)
            pltpu.async_copy(dst_hbm.at[sid, j], dring.at[slot], sem_i)

        def wait_pair(slot):
            for ring in (sring, dring):
                pltpu.make_async_copy(src_hbm.at[sid, 0], ring.at[slot],
                                      sem_i).wait()

        def start_gather(islot, rslot):
            pltpu.async_copy(y_sh.at[sring.at[islot]], rows_v.at[rslot],
                             sem_g)

        def wait_gather(rslot):
            pltpu.make_async_copy(ya_hbm.at[sring.at[0]], rows_v.at[rslot],
                                  sem_g).wait()

        def start_scatter(islot, rslot):
            pltpu.async_copy(rows_v.at[rslot], acc_sh.at[dring.at[islot]],
                             sem_s, add=True)

        def wait_scatter(rslot):
            pltpu.make_async_copy(rows_v.at[rslot], acc_sh.at[dring.at[0]],
                                  sem_s).wait()

        def count(islot, count_core):
            if with_counts:
                @pl.when(cid == count_core)
                def _():
                    pltpu.sync_copy(ones_v, cnt_sh.at[dring.at[islot]],
                                    add=True)

        for j in range(6):
            load_pair(j, j)

        plsc.subcore_barrier()

        for j in range(2):
            wait_pair(j)
            start_gather(j, j)

        for j in range(8):
            b8, b4 = j % 8, j % 4
            wait_gather(b4)
            start_scatter(b8, b4)
            count(b8, b4 % NC)
            if j >= 2:
                wait_scatter((b4 + 2) % 4)
            wait_pair((b8 + 2) % IRING)
            start_gather((b8 + 2) % IRING, (b4 + 2) % 4)
            load_pair(j + 6, (b8 + 6) % IRING)

        @pl.loop(8, nch - 8, step=8)
        def _(j0):
            for b in range(8):
                j = j0 + b
                b8, b4 = b % 8, b % 4
                wait_gather(b4)
                start_scatter(b8, b4)
                count(b8, b4 % NC)
                wait_scatter((b4 + 2) % 4)
                wait_pair((b8 + 2) % IRING)
                start_gather((b8 + 2) % IRING, (b4 + 2) % 4)
                load_pair(j + 6, (b8 + 6) % IRING)

        for b in range(8):
            b8, b4 = b % 8, b % 4
            wait_gather(b4)
            start_scatter(b8, b4)
            count(b8, b4 % NC)
            wait_scatter((b4 + 2) % 4)
            if b < 6:
                wait_pair((b8 + 2) % IRING)
                start_gather((b8 + 2) % IRING, (b4 + 2) % 4)
            if b < 2:
                load_pair(nch - 8 + b + 6, (b8 + 6) % IRING)
        wait_scatter(2)
        wait_scatter(3)

        plsc.subcore_barrier()
        pltpu.sync_copy(acc_sh.at[pl.ds(sid * rpt, rpt)],
                        part_out.at[cid, pl.ds(sid * rpt, rpt)])
        if with_counts:
            @pl.when(sid == 1)
            def _():
                pltpu.sync_copy(cnt_sh, cnt_out.at[cid])

    return k


def _tc1_body(x_ref, tau_ref, wl_ref, wr_ref, b_ref, ya_ref, yb_ref, yr_ref):
    xb = x_ref[...]
    t = tau_ref[...]
    d = xb.shape[1]
    hh = ya_ref.shape[1]
    yl = (jnp.dot(xb, wl_ref[:d, :], preferred_element_type=jnp.float32)
          + t * wl_ref[d:d + 1, :])
    ya_ref[...] = yl[:, :hh]
    yb_ref[...] = yl[:, hh:]
    yr_ref[...] = (jnp.dot(xb, wr_ref[:d, :], preferred_element_type=jnp.float32)
                   + t * wr_ref[d:d + 1, :] + b_ref[...])


def _tc2_body(p_ref, cnt_ref, yr_ref, wl_ref, wr_ref, b_ref,
              y2a_ref, y2b_ref, y2r_ref, inv_ref):
    s = jnp.concatenate([p_ref[0], p_ref[1]], axis=1)
    hh = y2a_ref.shape[1]
    cnt = cnt_ref[0, :, 0] + cnt_ref[1, :, 0]
    inv = 1.0 / jnp.clip(cnt, 1.0, None)
    h1 = jnp.maximum(s * inv[:, None] + yr_ref[...], 0.0)
    y2l = jnp.dot(h1, wl_ref[...], preferred_element_type=jnp.float32)
    y2a_ref[...] = y2l[:, :hh]
    y2b_ref[...] = y2l[:, hh:]
    y2r_ref[...] = (jnp.dot(h1, wr_ref[...], preferred_element_type=jnp.float32)
                    + b_ref[...])
    inv_ref[...] = inv[:, None]


def _tc3_body(p_ref, inv_ref, yr_ref, wf_ref, bf_ref, out_ref):
    s = jnp.concatenate([p_ref[0], p_ref[1]], axis=1)
    h2 = jnp.maximum(s * inv_ref[...] + yr_ref[...], 0.0)
    out_ref[...] = (jnp.dot(h2, wf_ref[...], preferred_element_type=jnp.float32)
                    + bf_ref[...])


def kernel(x, edge_index, tau, W1l, W1r, b1, W2l, W2r, b2, Wf, bf):
    n, d = x.shape
    h = W1l.shape[1]
    hh = h // 2
    e = edge_index.shape[1]

    ept = -(-e // NS)
    nch = -(-ept // CHUNK)
    nch = -(-nch // 8) * 8
    ep = NS * nch * CHUNK
    n_pad = -(-(n + 1) // (NS * 8)) * (NS * 8)
    rpt = n_pad // NS

    src = jnp.concatenate([edge_index[0], jnp.zeros((ep - e,), jnp.int32)])
    dst = jnp.concatenate([edge_index[1], jnp.full((ep - e,), n, jnp.int32)])
    src3 = src.reshape(NS, nch, CHUNK)
    dst3 = dst.reshape(NS, nch, CHUNK)
    zeros2 = jnp.zeros((rpt, hh), jnp.float32)
    zeros1 = jnp.zeros((n_pad,), jnp.float32)

    seg1 = _make_seg_kernel(n, n_pad, hh, nch, True)
    seg2 = _make_seg_kernel(n, n_pad, hh, nch, False)

    r = n if n % 8 == 0 else 8 * (-(-n // 8))
    grid = n // r if n % 8 == 0 else 1
    full = lambda i: (0, 0)
    row2 = pl.BlockSpec((r, h), lambda i: (i, 0))
    rowh = pl.BlockSpec((r, hh), lambda i: (i, 0))
    col1 = pl.BlockSpec((r, 1), lambda i: (i, 0))
    part_spec = pl.BlockSpec((NC, r, hh), lambda i: (0, i, 0))

    tc1 = pl.pallas_call(
        _tc1_body, grid=(grid,),
        in_specs=[pl.BlockSpec((r, d), lambda i: (i, 0)),
                  col1,
                  pl.BlockSpec((d + 1, h), full),
                  pl.BlockSpec((d + 1, h), full),
                  pl.BlockSpec((1, h), full)],
        out_specs=[rowh, rowh, row2],
        out_shape=[jax.ShapeDtypeStruct((n, hh), jnp.float32),
                   jax.ShapeDtypeStruct((n, hh), jnp.float32),
                   jax.ShapeDtypeStruct((n, h), jnp.float32)],
    )
    tc2 = pl.pallas_call(
        _tc2_body, grid=(grid,),
        in_specs=[part_spec,
                  pl.BlockSpec((NC, r, 1), lambda i: (0, i, 0)),
                  row2,
                  pl.BlockSpec((h, h), full),
                  pl.BlockSpec((h, h), full),
                  pl.BlockSpec((1, h), full)],
        out_specs=[rowh, rowh, row2, col1],
        out_shape=[jax.ShapeDtypeStruct((n, hh), jnp.float32),
                   jax.ShapeDtypeStruct((n, hh), jnp.float32),
                   jax.ShapeDtypeStruct((n, h), jnp.float32),
                   jax.ShapeDtypeStruct((n, 1), jnp.float32)],
    )
    tc3 = pl.pallas_call(
        _tc3_body, grid=(grid,),
        in_specs=[part_spec,
                  col1,
                  row2,
                  pl.BlockSpec((h, 1), full),
                  pl.BlockSpec((1, 1), full)],
        out_specs=col1,
        out_shape=jax.ShapeDtypeStruct((n, 1), jnp.float32),
    )

    y1a, y1b, y1r = tc1(x, tau, W1l, W1r, b1.reshape(1, h))
    part1, cnt = seg1(y1a, y1b, src3, dst3, zeros2, zeros1)
    y2a, y2b, y2r, inv = tc2(part1, cnt.reshape(NC, n_pad, 1), y1r,
                             W2l, W2r, b2.reshape(1, h))
    (part2,) = seg2(y2a, y2b, src3, dst3, zeros2, zeros1)
    return tc3(part2, inv, y2r, Wf, bf.reshape(1, 1))


# speedup vs baseline: 10.3668x; 1.0946x over previous
import functools

import jax
import jax.numpy as jnp
from jax import lax
from jax.experimental import pallas as pl
from jax.experimental.pallas import tpu as pltpu
from jax.experimental.pallas import tpu_sc as plsc

NC = 2
NS = 16
LANES = 16
CHUNK = 128
RING = 4
IRING = 8


@functools.lru_cache(maxsize=None)
def _make_seg_kernel(n_nodes, n_pad, hh, nch, with_counts):
    rpt = n_pad // NS
    mesh = plsc.VectorSubcoreMesh(
        core_axis_name="c", subcore_axis_name="s",
        num_cores=NC, num_subcores=NS)

    out_type = [jax.ShapeDtypeStruct((NC, n_pad, hh), jnp.float32)]
    if with_counts:
        out_type.append(jax.ShapeDtypeStruct((NC, n_pad), jnp.float32))

    scratch = [
        pltpu.VMEM((IRING, CHUNK), jnp.int32),
        pltpu.VMEM((IRING, CHUNK), jnp.int32),
        pltpu.VMEM((RING, CHUNK, hh), jnp.float32),
        pltpu.VMEM_SHARED((n_nodes, hh), jnp.float32),
        pltpu.VMEM_SHARED((n_pad, hh), jnp.float32),
        pltpu.SemaphoreType.DMA,
        pltpu.SemaphoreType.DMA,
        pltpu.SemaphoreType.DMA,
    ]
    if with_counts:
        scratch.insert(3, pltpu.VMEM((CHUNK,), jnp.float32))
        scratch.insert(4, pltpu.VMEM_SHARED((n_pad,), jnp.float32))
        scratch.append(pltpu.SemaphoreType.DMA)

    @functools.partial(
        pl.kernel, out_type=tuple(out_type), mesh=mesh,
        scratch_types=tuple(scratch),
        compiler_params=pltpu.CompilerParams(needs_layout_passes=False,
                                             use_tc_tiling_on_sc=False))
    def k(*refs):
        if with_counts:
            (ya_hbm, yb_hbm, src_hbm, dst_hbm, zeros2_hbm, zeros1_hbm,
             part_out, cnt_out,
             sring, dring, rows_v, ones_v, cnt_sh, y_sh, acc_sh,
             sem_g, sem_i, sem_s, sem_c) = refs
        else:
            (ya_hbm, yb_hbm, src_hbm, dst_hbm, zeros2_hbm, zeros1_hbm,
             part_out,
             sring, dring, rows_v, y_sh, acc_sh, sem_g, sem_i, sem_s) = refs
        cid = lax.axis_index("c")
        sid = lax.axis_index("s")

        @pl.when(sid == 0)
        def _():
            @pl.when(cid == 0)
            def _():
                pltpu.sync_copy(ya_hbm, y_sh)

            @pl.when(cid == 1)
            def _():
                pltpu.sync_copy(yb_hbm, y_sh)

        pltpu.sync_copy(zeros2_hbm, acc_sh.at[pl.ds(sid * rpt, rpt)])

        if with_counts:
            @pl.when(sid == 1)
            def _():
                pltpu.sync_copy(zeros1_hbm, cnt_sh)
            one16 = jnp.ones((LANES,), jnp.float32)

            def obody(i, c):
                ones_v[pl.ds(i * LANES, LANES)] = one16
                return c
            lax.fori_loop(0, CHUNK // LANES, obody, 0)

        def load_pair(j, slot):
            pltpu.async_copy(src_hbm.at[sid, j], sring.at[slot], sem_i)
            pltpu.async_copy(dst_hbm.at[sid, j], dring.at[slot], sem_i)

        def wait_pair(slot):
            for ring in (sring, dring):
                pltpu.make_async_copy(src_hbm.at[sid, 0], ring.at[slot],
                                      sem_i).wait()

        def start_gather(islot, rslot):
            pltpu.async_copy(y_sh.at[sring.at[islot]], rows_v.at[rslot],
                             sem_g)

        def wait_gather(rslot):
            pltpu.make_async_copy(ya_hbm.at[sring.at[0]], rows_v.at[rslot],
                                  sem_g).wait()

        def start_scatter(islot, rslot):
            pltpu.async_copy(rows_v.at[rslot], acc_sh.at[dring.at[islot]],
                             sem_s, add=True)

        def wait_scatter(rslot):
            pltpu.make_async_copy(rows_v.at[rslot], acc_sh.at[dring.at[0]],
                                  sem_s).wait()

        def count(islot, count_core):
            if with_counts:
                @pl.when(cid == count_core)
                def _():
                    pltpu.async_copy(ones_v, cnt_sh.at[dring.at[islot]],
                                     sem_c, add=True)

        def wait_count(count_core):
            if with_counts:
                @pl.when(cid == count_core)
                def _():
                    pltpu.make_async_copy(ones_v, cnt_sh.at[dring.at[0]],
                                          sem_c).wait()

        for j in range(6):
            load_pair(j, j)

        plsc.subcore_barrier()

        for j in range(2):
            wait_pair(j)
            start_gather(j, j)

        for j in range(8):
            b8, b4 = j % 8, j % 4
            wait_gather(b4)
            start_scatter(b8, b4)
            count(b8, b4 % NC)
            if j >= 2:
                wait_scatter((b4 + 2) % 4)
                wait_count(b4 % NC)
            wait_pair((b8 + 2) % IRING)
            start_gather((b8 + 2) % IRING, (b4 + 2) % 4)
            load_pair(j + 6, (b8 + 6) % IRING)

        @pl.loop(8, nch - 8, step=8)
        def _(j0):
            for b in range(8):
                j = j0 + b
                b8, b4 = b % 8, b % 4
                wait_gather(b4)
                start_scatter(b8, b4)
                count(b8, b4 % NC)
                wait_scatter((b4 + 2) % 4)
                wait_count(b4 % NC)
                wait_pair((b8 + 2) % IRING)
                start_gather((b8 + 2) % IRING, (b4 + 2) % 4)
                load_pair(j + 6, (b8 + 6) % IRING)

        for b in range(8):
            b8, b4 = b % 8, b % 4
            wait_gather(b4)
            start_scatter(b8, b4)
            count(b8, b4 % NC)
            wait_scatter((b4 + 2) % 4)
            wait_count(b4 % NC)
            if b < 6:
                wait_pair((b8 + 2) % IRING)
                start_gather((b8 + 2) % IRING, (b4 + 2) % 4)
            if b < 2:
                load_pair(nch - 8 + b + 6, (b8 + 6) % IRING)
        wait_scatter(2)
        wait_scatter(3)
        wait_count(0)
        wait_count(1)

        plsc.subcore_barrier()
        pltpu.sync_copy(acc_sh.at[pl.ds(sid * rpt, rpt)],
                        part_out.at[cid, pl.ds(sid * rpt, rpt)])
        if with_counts:
            @pl.when(sid == 1)
            def _():
                pltpu.sync_copy(cnt_sh, cnt_out.at[cid])

    return k


def _tc1_body(x_ref, tau_ref, wl_ref, wr_ref, b_ref, ya_ref, yb_ref, yr_ref):
    xb = x_ref[...]
    t = tau_ref[...]
    d = xb.shape[1]
    hh = ya_ref.shape[1]
    yl = (jnp.dot(xb, wl_ref[:d, :], preferred_element_type=jnp.float32)
          + t * wl_ref[d:d + 1, :])
    ya_ref[...] = yl[:, :hh]
    yb_ref[...] = yl[:, hh:]
    yr_ref[...] = (jnp.dot(xb, wr_ref[:d, :], preferred_element_type=jnp.float32)
                   + t * wr_ref[d:d + 1, :] + b_ref[...])


def _tc2_body(p_ref, cnt_ref, yr_ref, wl_ref, wr_ref, b_ref,
              y2a_ref, y2b_ref, y2r_ref, inv_ref):
    s = jnp.concatenate([p_ref[0], p_ref[1]], axis=1)
    hh = y2a_ref.shape[1]
    cnt = cnt_ref[0, :, 0] + cnt_ref[1, :, 0]
    inv = 1.0 / jnp.clip(cnt, 1.0, None)
    h1 = jnp.maximum(s * inv[:, None] + yr_ref[...], 0.0)
    y2l = jnp.dot(h1, wl_ref[...], preferred_element_type=jnp.float32)
    y2a_ref[...] = y2l[:, :hh]
    y2b_ref[...] = y2l[:, hh:]
    y2r_ref[...] = (jnp.dot(h1, wr_ref[...], preferred_element_type=jnp.float32)
                    + b_ref[...])
    inv_ref[...] = inv[:, None]


def _tc3_body(p_ref, inv_ref, yr_ref, wf_ref, bf_ref, out_ref):
    s = jnp.concatenate([p_ref[0], p_ref[1]], axis=1)
    h2 = jnp.maximum(s * inv_ref[...] + yr_ref[...], 0.0)
    out_ref[...] = (jnp.dot(h2, wf_ref[...], preferred_element_type=jnp.float32)
                    + bf_ref[...])


def kernel(x, edge_index, tau, W1l, W1r, b1, W2l, W2r, b2, Wf, bf):
    n, d = x.shape
    h = W1l.shape[1]
    hh = h // 2
    e = edge_index.shape[1]

    ept = -(-e // NS)
    nch = -(-ept // CHUNK)
    nch = -(-nch // 8) * 8
    ep = NS * nch * CHUNK
    n_pad = -(-(n + 1) // (NS * 8)) * (NS * 8)
    rpt = n_pad // NS

    src = jnp.concatenate([edge_index[0], jnp.zeros((ep - e,), jnp.int32)])
    dst = jnp.concatenate([edge_index[1], jnp.full((ep - e,), n, jnp.int32)])
    src3 = src.reshape(NS, nch, CHUNK)
    dst3 = dst.reshape(NS, nch, CHUNK)
    zeros2 = jnp.zeros((rpt, hh), jnp.float32)
    zeros1 = jnp.zeros((n_pad,), jnp.float32)

    seg1 = _make_seg_kernel(n, n_pad, hh, nch, True)
    seg2 = _make_seg_kernel(n, n_pad, hh, nch, False)

    r = n if n % 8 == 0 else 8 * (-(-n // 8))
    grid = n // r if n % 8 == 0 else 1
    full = lambda i: (0, 0)
    row2 = pl.BlockSpec((r, h), lambda i: (i, 0))
    rowh = pl.BlockSpec((r, hh), lambda i: (i, 0))
    col1 = pl.BlockSpec((r, 1), lambda i: (i, 0))
    part_spec = pl.BlockSpec((NC, r, hh), lambda i: (0, i, 0))

    tc1 = pl.pallas_call(
        _tc1_body, grid=(grid,),
        in_specs=[pl.BlockSpec((r, d), lambda i: (i, 0)),
                  col1,
                  pl.BlockSpec((d + 1, h), full),
                  pl.BlockSpec((d + 1, h), full),
                  pl.BlockSpec((1, h), full)],
        out_specs=[rowh, rowh, row2],
        out_shape=[jax.ShapeDtypeStruct((n, hh), jnp.float32),
                   jax.ShapeDtypeStruct((n, hh), jnp.float32),
                   jax.ShapeDtypeStruct((n, h), jnp.float32)],
    )
    tc2 = pl.pallas_call(
        _tc2_body, grid=(grid,),
        in_specs=[part_spec,
                  pl.BlockSpec((NC, r, 1), lambda i: (0, i, 0)),
                  row2,
                  pl.BlockSpec((h, h), full),
                  pl.BlockSpec((h, h), full),
                  pl.BlockSpec((1, h), full)],
        out_specs=[rowh, rowh, row2, col1],
        out_shape=[jax.ShapeDtypeStruct((n, hh), jnp.float32),
                   jax.ShapeDtypeStruct((n, hh), jnp.float32),
                   jax.ShapeDtypeStruct((n, h), jnp.float32),
                   jax.ShapeDtypeStruct((n, 1), jnp.float32)],
    )
    tc3 = pl.pallas_call(
        _tc3_body, grid=(grid,),
        in_specs=[part_spec,
                  col1,
                  row2,
                  pl.BlockSpec((h, 1), full),
                  pl.BlockSpec((1, 1), full)],
        out_specs=col1,
        out_shape=jax.ShapeDtypeStruct((n, 1), jnp.float32),
    )

    y1a, y1b, y1r = tc1(x, tau, W1l, W1r, b1.reshape(1, h))
    part1, cnt = seg1(y1a, y1b, src3, dst3, zeros2, zeros1)
    y2a, y2b, y2r, inv = tc2(part1, cnt.reshape(NC, n_pad, 1), y1r,
                             W2l, W2r, b2.reshape(1, h))
    (part2,) = seg2(y2a, y2b, src3, dst3, zeros2, zeros1)
    return tc3(part2, inv, y2r, Wf, bf.reshape(1, 1))
